# baseline (speedup 1.0000x reference)
import functools
import math

import jax
import jax.numpy as jnp
import numpy as np
from jax import lax
from jax.experimental import pallas as pl
from jax.experimental.pallas import tpu as pltpu
from jax.sharding import Mesh, PartitionSpec as P

_NEG_INF = -1e30


def _mhsa_kernel(x_ref, wqkv_ref, bqkv_ref, wo_ref, bo_ref, o_ref, *,
                 n_heads, scale):
    S = x_ref.shape[1]
    D = x_ref.shape[2]
    dh = D // n_heads

    x = x_ref[0].astype(jnp.bfloat16)
    qkv = lax.dot_general(
        x, wqkv_ref[...], (((1,), (1,)), ((), ())),
        preferred_element_type=jnp.float32) + bqkv_ref[...]

    qi = lax.broadcasted_iota(jnp.int32, (S, S), 0)
    ki = lax.broadcasted_iota(jnp.int32, (S, S), 1)
    causal = ki <= qi

    heads = []
    for h in range(n_heads):
        q = (qkv[:, h * dh:(h + 1) * dh] * scale).astype(jnp.bfloat16)
        k = qkv[:, D + h * dh:D + (h + 1) * dh].astype(jnp.bfloat16)
        v = qkv[:, 2 * D + h * dh:2 * D + (h + 1) * dh].astype(jnp.bfloat16)
        s = lax.dot_general(q, k, (((1,), (1,)), ((), ())),
                            preferred_element_type=jnp.float32)
        s = jnp.where(causal, s, _NEG_INF)
        m = jnp.max(s, axis=-1, keepdims=True)
        p = jnp.exp(s - m)
        l = jnp.sum(p, axis=-1, keepdims=True)
        o = lax.dot_general(p.astype(jnp.bfloat16), v,
                            (((1,), (0,)), ((), ())),
                            preferred_element_type=jnp.float32)
        heads.append((o / l).astype(jnp.bfloat16))

    attn = jnp.concatenate(heads, axis=1)
    out = lax.dot_general(attn, wo_ref[...], (((1,), (1,)), ((), ())),
                          preferred_element_type=jnp.float32) + bo_ref[...]
    o_ref[0] = out.astype(o_ref.dtype)


def _mhsa_call(x, w_qkv, b_qkv, wo, bo, *, n_heads, scale):
    B, S, D = x.shape
    return pl.pallas_call(
        functools.partial(_mhsa_kernel, n_heads=n_heads, scale=scale),
        out_shape=jax.ShapeDtypeStruct((B, S, D), x.dtype),
        grid=(B,),
        in_specs=[
            pl.BlockSpec((1, S, D), lambda b: (b, 0, 0)),
            pl.BlockSpec((3 * D, D), lambda b: (0, 0)),
            pl.BlockSpec((1, 3 * D), lambda b: (0, 0)),
            pl.BlockSpec((D, D), lambda b: (0, 0)),
            pl.BlockSpec((1, D), lambda b: (0, 0)),
        ],
        out_specs=pl.BlockSpec((1, S, D), lambda b: (b, 0, 0)),
        compiler_params=pltpu.CompilerParams(
            dimension_semantics=("parallel",),
            vmem_limit_bytes=(56 << 20)),
    )(x, w_qkv, b_qkv, wo, bo)


def kernel(x, w_in, b_in, w_out, b_out):
    B, S, D = x.shape
    H = 12
    dh = D // H
    scale = 1.0 / math.sqrt(dh)

    w_qkv = w_in.astype(jnp.bfloat16)
    b_qkv = b_in.reshape(1, 3 * D)
    wo = w_out.astype(jnp.bfloat16)
    bo = b_out.reshape(1, D)

    call = functools.partial(_mhsa_call, n_heads=H, scale=scale)

    devs = jax.devices()
    if len(devs) >= 2 and B % 2 == 0:
        mesh = Mesh(np.asarray(devs[:2]), ("d",))
        sharded = jax.shard_map(
            call, mesh=mesh,
            in_specs=(P("d"), P(), P(), P(), P()),
            out_specs=P("d"), check_vma=False)
        return sharded(x, w_qkv, b_qkv, wo, bo)
    return call(x, w_qkv, b_qkv, wo, bo)


# speedup vs baseline: 1.9902x; 1.9902x over previous
import functools
import math

import jax
import jax.numpy as jnp
from jax import lax
from jax.experimental import pallas as pl
from jax.experimental.pallas import tpu as pltpu

_NEG_INF = -1e30
_N_Q_CHUNKS = 4


def _mhsa_kernel(x_ref, wqkv_ref, bqkv_ref, wo_ref, bo_ref, o_ref, *,
                 n_heads, scale):
    S = x_ref.shape[1]
    D = x_ref.shape[2]
    dh = D // n_heads
    nq = _N_Q_CHUNKS
    tq = S // nq

    x = x_ref[0].astype(jnp.bfloat16)
    qkv = lax.dot_general(
        x, wqkv_ref[...], (((1,), (1,)), ((), ())),
        preferred_element_type=jnp.float32) + bqkv_ref[...]

    q_bf, k_bf, v_bf = [], [], []
    for h in range(n_heads):
        q_bf.append((qkv[:, h * dh:(h + 1) * dh] * scale).astype(jnp.bfloat16))
        k_bf.append(qkv[:, D + h * dh:D + (h + 1) * dh].astype(jnp.bfloat16))
        v_bf.append(qkv[:, 2 * D + h * dh:2 * D + (h + 1) * dh]
                    .astype(jnp.bfloat16))

    masks = []
    for i in range(nq):
        kv_len = (i + 1) * tq
        qi = i * tq + lax.broadcasted_iota(jnp.int32, (tq, kv_len), 0)
        ki = lax.broadcasted_iota(jnp.int32, (tq, kv_len), 1)
        masks.append(ki <= qi)

    for i in range(nq):
        kv_len = (i + 1) * tq
        pieces = []
        for h in range(n_heads):
            q = q_bf[h][i * tq:(i + 1) * tq]
            k = k_bf[h][:kv_len]
            v = v_bf[h][:kv_len]
            s = lax.dot_general(q, k, (((1,), (1,)), ((), ())),
                                preferred_element_type=jnp.float32)
            s = jnp.where(masks[i], s, _NEG_INF)
            m = jnp.max(s, axis=-1, keepdims=True)
            p = jnp.exp(s - m)
            l = jnp.sum(p, axis=-1, keepdims=True)
            o = lax.dot_general(p.astype(jnp.bfloat16), v,
                                (((1,), (0,)), ((), ())),
                                preferred_element_type=jnp.float32)
            pieces.append((o / l).astype(jnp.bfloat16))
        attn = jnp.concatenate(pieces, axis=1)
        out = lax.dot_general(attn, wo_ref[...], (((1,), (1,)), ((), ())),
                              preferred_element_type=jnp.float32) + bo_ref[...]
        o_ref[0, i * tq:(i + 1) * tq, :] = out.astype(o_ref.dtype)


def kernel(x, w_in, b_in, w_out, b_out):
    B, S, D = x.shape
    H = 12
    dh = D // H
    scale = 1.0 / math.sqrt(dh)

    w_qkv = w_in.astype(jnp.bfloat16)
    b_qkv = b_in.reshape(1, 3 * D)
    wo = w_out.astype(jnp.bfloat16)
    bo = b_out.reshape(1, D)

    return pl.pallas_call(
        functools.partial(_mhsa_kernel, n_heads=H, scale=scale),
        out_shape=jax.ShapeDtypeStruct((B, S, D), x.dtype),
        grid=(B,),
        in_specs=[
            pl.BlockSpec((1, S, D), lambda b: (b, 0, 0)),
            pl.BlockSpec((3 * D, D), lambda b: (0, 0)),
            pl.BlockSpec((1, 3 * D), lambda b: (0, 0)),
            pl.BlockSpec((D, D), lambda b: (0, 0)),
            pl.BlockSpec((1, D), lambda b: (0, 0)),
        ],
        out_specs=pl.BlockSpec((1, S, D), lambda b: (b, 0, 0)),
        compiler_params=pltpu.CompilerParams(
            dimension_semantics=("parallel",),
            vmem_limit_bytes=(56 << 20)),
    )(x, w_qkv, b_qkv, wo, bo)


# speedup vs baseline: 3.2736x; 1.6449x over previous
import functools
import math

import jax
import jax.numpy as jnp
from jax import lax
from jax.experimental import pallas as pl
from jax.experimental.pallas import tpu as pltpu


def _mhsa_kernel(x_ref, wqkv_ref, bqkv_ref, wo_ref, bo_ref, o_ref, *,
                 n_heads, scale):
    S = x_ref.shape[1]
    D = x_ref.shape[2]
    dh = D // n_heads

    x = x_ref[0].astype(jnp.bfloat16)
    qkv = lax.dot_general(
        x, wqkv_ref[...], (((1,), (1,)), ((), ())),
        preferred_element_type=jnp.float32) + bqkv_ref[...]

    qi = lax.broadcasted_iota(jnp.int32, (S, S), 0)
    ki = lax.broadcasted_iota(jnp.int32, (S, S), 1)
    causal01 = (ki <= qi).astype(jnp.float32)

    heads = []
    for h in range(n_heads):
        q = (qkv[:, h * dh:(h + 1) * dh] * scale).astype(jnp.bfloat16)
        k = qkv[:, D + h * dh:D + (h + 1) * dh].astype(jnp.bfloat16)
        v = qkv[:, 2 * D + h * dh:2 * D + (h + 1) * dh].astype(jnp.bfloat16)
        s = lax.dot_general(q, k, (((1,), (1,)), ((), ())),
                            preferred_element_type=jnp.float32)
        m = jnp.max(s, axis=-1, keepdims=True)
        p = jnp.exp(s - m) * causal01
        l = jnp.sum(p, axis=-1, keepdims=True)
        o = lax.dot_general(p.astype(jnp.bfloat16), v,
                            (((1,), (0,)), ((), ())),
                            preferred_element_type=jnp.float32)
        heads.append((o / l).astype(jnp.bfloat16))

    attn = jnp.concatenate(heads, axis=1)
    out = lax.dot_general(attn, wo_ref[...], (((1,), (1,)), ((), ())),
                          preferred_element_type=jnp.float32) + bo_ref[...]
    o_ref[0] = out.astype(o_ref.dtype)


def kernel(x, w_in, b_in, w_out, b_out):
    B, S, D = x.shape
    H = 12
    dh = D // H
    scale = 1.0 / math.sqrt(dh)

    w_qkv = w_in.astype(jnp.bfloat16)
    b_qkv = b_in.reshape(1, 3 * D)
    wo = w_out.astype(jnp.bfloat16)
    bo = b_out.reshape(1, D)

    return pl.pallas_call(
        functools.partial(_mhsa_kernel, n_heads=H, scale=scale),
        out_shape=jax.ShapeDtypeStruct((B, S, D), x.dtype),
        grid=(B,),
        in_specs=[
            pl.BlockSpec((1, S, D), lambda b: (b, 0, 0)),
            pl.BlockSpec((3 * D, D), lambda b: (0, 0)),
            pl.BlockSpec((1, 3 * D), lambda b: (0, 0)),
            pl.BlockSpec((D, D), lambda b: (0, 0)),
            pl.BlockSpec((1, D), lambda b: (0, 0)),
        ],
        out_specs=pl.BlockSpec((1, S, D), lambda b: (b, 0, 0)),
        compiler_params=pltpu.CompilerParams(
            dimension_semantics=("parallel",),
            vmem_limit_bytes=(56 << 20)),
    )(x, w_qkv, b_qkv, wo, bo)
